# baseline (speedup 1.0000x reference)
import jax
import jax.numpy as jnp
from jax.experimental import pallas as pl
from jax.experimental.pallas import tpu as pltpu

_NE = 512
_D = 64
_B = 64
_N = 1024
_NTOK = _B * _N
_IB = 8


def _vq_block(z_ref, w_ref, q_ref, idx_ref, perp_ref, uniq_ref,
              sww_ref, wm2_ref, wbf_ref, counts_ref):
    i = pl.program_id(0)
    nb = pl.num_programs(0)

    @pl.when(i == 0)
    def _prep():
        w = w_ref[...]
        sww_ref[...] = jax.lax.dot_general(
            jnp.ones((1, _D), jnp.float32), w * w, (((1,), (1,)), ((), ())))
        wm2_ref[...] = w * (-2.0)
        wbf_ref[...] = w.astype(jnp.bfloat16)
        counts_ref[...] = jnp.zeros_like(counts_ref)

    zb = z_ref[...].reshape(_N, _D)

    szz = jnp.sum(zb * zb, axis=1, keepdims=True)
    scores2 = jax.lax.dot_general(zb, wm2_ref[...],
                                  (((1,), (1,)), ((), ())))
    dist = (szz + sww_ref[...]) + scores2

    dmin = jnp.min(dist, axis=1, keepdims=True)
    lane = jax.lax.broadcasted_iota(jnp.int32, dist.shape, 1)
    idx = jnp.min(jnp.where(dist == dmin, lane, _NE),
                  axis=1, keepdims=True)

    onehot = (lane == idx).astype(jnp.bfloat16)
    qb = jax.lax.dot_general(onehot, wbf_ref[...], (((1,), (0,)), ((), ())),
                             preferred_element_type=jnp.float32)
    q_ref[...] = (zb + (qb - zb)).reshape(1, _N, _D)
    idx_ref[pl.ds(jax.lax.rem(i, _IB), 1), :] = idx.reshape(1, _N)

    counts_ref[...] += jax.lax.dot_general(
        jnp.ones((1, _N), jnp.bfloat16), onehot, (((1,), (0,)), ((), ())),
        preferred_element_type=jnp.float32)

    @pl.when(i == nb - 1)
    def _fin():
        counts = counts_ref[...]
        avg = counts * (1.0 / _NTOK)
        ent = jnp.sum(avg * jnp.log(avg + 1e-10), axis=(0, 1), keepdims=True)
        perp_ref[...] = jnp.exp(-ent)
        uniq_ref[...] = jnp.sum((counts > 0.0).astype(jnp.int32),
                                axis=(0, 1), keepdims=True)


def kernel(z, W):
    original_dtype = z.dtype
    zf = z.astype(jnp.float32)
    q, idx, perp, uniq = pl.pallas_call(
        _vq_block,
        grid=(_B,),
        in_specs=[
            pl.BlockSpec((1, _N, _D), lambda i: (i, 0, 0)),
            pl.BlockSpec((_NE, _D), lambda i: (0, 0)),
        ],
        out_specs=[
            pl.BlockSpec((1, _N, _D), lambda i: (i, 0, 0)),
            pl.BlockSpec((_IB, _N), lambda i: (i // _IB, 0)),
            pl.BlockSpec((1, 1), lambda i: (0, 0)),
            pl.BlockSpec((1, 1), lambda i: (0, 0)),
        ],
        out_shape=[
            jax.ShapeDtypeStruct((_B, _N, _D), jnp.float32),
            jax.ShapeDtypeStruct((_B, _N), jnp.int32),
            jax.ShapeDtypeStruct((1, 1), jnp.float32),
            jax.ShapeDtypeStruct((1, 1), jnp.int32),
        ],
        scratch_shapes=[
            pltpu.VMEM((1, _NE), jnp.float32),
            pltpu.VMEM((_NE, _D), jnp.float32),
            pltpu.VMEM((_NE, _D), jnp.bfloat16),
            pltpu.VMEM((1, _NE), jnp.float32),
        ],
        compiler_params=pltpu.CompilerParams(
            dimension_semantics=("arbitrary",)),
    )(zf, W)
    quantized = q.astype(original_dtype)
    vq_loss = jnp.zeros((), jnp.float32)
    return (quantized, idx, vq_loss, perp.reshape(()), uniq.reshape(()))


# speedup vs baseline: 1.1620x; 1.1620x over previous
import jax
import jax.numpy as jnp
from jax.experimental import pallas as pl
from jax.experimental.pallas import tpu as pltpu

_NE = 512
_D = 64
_B = 64
_N = 1024
_NTOK = _B * _N
_RB = 2
_R = _RB * _N
_NB = _B // _RB
_IB = 8


def _vq_block(z_ref, w_ref, q_ref, idx_ref, perp_ref, uniq_ref,
              sww_ref, wm2_ref, wf_ref, counts_ref):
    i = pl.program_id(0)
    nb = pl.num_programs(0)

    @pl.when(i == 0)
    def _prep():
        w = w_ref[...]
        sww_ref[...] = jax.lax.dot_general(
            jnp.ones((1, _D), jnp.float32), w * w, (((1,), (1,)), ((), ())))
        wm2_ref[...] = w * (-2.0)
        wf_ref[...] = w
        counts_ref[...] = jnp.zeros_like(counts_ref)

    zb = z_ref[...].reshape(_R, _D)

    szz = jnp.sum(zb * zb, axis=1, keepdims=True)
    scores2 = jax.lax.dot_general(zb, wm2_ref[...],
                                  (((1,), (1,)), ((), ())))
    dist = (szz + sww_ref[...]) + scores2

    dmin = jnp.min(dist, axis=1, keepdims=True)
    lane = jax.lax.broadcasted_iota(jnp.int32, dist.shape, 1)
    idx = jnp.min(jnp.where(dist == dmin, lane, _NE),
                  axis=1, keepdims=True)

    onehot = (lane == idx).astype(jnp.float32)
    qb = jax.lax.dot_general(onehot, wf_ref[...], (((1,), (0,)), ((), ())),
                             preferred_element_type=jnp.float32)
    q_ref[...] = (zb + (qb - zb)).reshape(_RB, _N, _D)

    iota_row = jax.lax.broadcasted_iota(
        jnp.int32, (1, _NE), 1).astype(jnp.float32)
    base = jax.lax.rem(i, _IB // _RB) * _RB
    for j in range(_RB):
        oh_j = jax.lax.slice(onehot, (j * _N, 0), ((j + 1) * _N, _NE))
        row = jax.lax.dot_general(iota_row, oh_j, (((1,), (1,)), ((), ())),
                                  preferred_element_type=jnp.float32)
        idx_ref[pl.ds(base + j, 1), :] = row.astype(jnp.int32)

    counts_ref[...] += jax.lax.dot_general(
        jnp.ones((1, _R), jnp.float32), onehot, (((1,), (0,)), ((), ())),
        preferred_element_type=jnp.float32)

    @pl.when(i == nb - 1)
    def _fin():
        counts = counts_ref[...]
        avg = counts * (1.0 / _NTOK)
        ent = jnp.sum(avg * jnp.log(avg + 1e-10), axis=(0, 1), keepdims=True)
        perp_ref[...] = jnp.exp(-ent)
        uniq_ref[...] = jnp.sum((counts > 0.0).astype(jnp.int32),
                                axis=(0, 1), keepdims=True)


def kernel(z, W):
    original_dtype = z.dtype
    zf = z.astype(jnp.float32)
    q, idx, perp, uniq = pl.pallas_call(
        _vq_block,
        grid=(_NB,),
        in_specs=[
            pl.BlockSpec((_RB, _N, _D), lambda i: (i, 0, 0)),
            pl.BlockSpec((_NE, _D), lambda i: (0, 0)),
        ],
        out_specs=[
            pl.BlockSpec((_RB, _N, _D), lambda i: (i, 0, 0)),
            pl.BlockSpec((_IB, _N), lambda i: (i // (_IB // _RB), 0)),
            pl.BlockSpec((1, 1), lambda i: (0, 0)),
            pl.BlockSpec((1, 1), lambda i: (0, 0)),
        ],
        out_shape=[
            jax.ShapeDtypeStruct((_B, _N, _D), jnp.float32),
            jax.ShapeDtypeStruct((_B, _N), jnp.int32),
            jax.ShapeDtypeStruct((1, 1), jnp.float32),
            jax.ShapeDtypeStruct((1, 1), jnp.int32),
        ],
        scratch_shapes=[
            pltpu.VMEM((1, _NE), jnp.float32),
            pltpu.VMEM((_NE, _D), jnp.float32),
            pltpu.VMEM((_NE, _D), jnp.float32),
            pltpu.VMEM((1, _NE), jnp.float32),
        ],
        compiler_params=pltpu.CompilerParams(
            dimension_semantics=("arbitrary",)),
    )(zf, W)
    quantized = q.astype(original_dtype)
    vq_loss = jnp.zeros((), jnp.float32)
    return (quantized, idx, vq_loss, perp.reshape(()), uniq.reshape(()))


# speedup vs baseline: 1.1950x; 1.0284x over previous
import jax
import jax.numpy as jnp
from jax.experimental import pallas as pl
from jax.experimental.pallas import tpu as pltpu

_NE = 512
_D = 64
_B = 64
_N = 1024
_NTOK = _B * _N
_RB = 4
_R = _RB * _N
_NB = _B // _RB
_IB = 8


def _vq_block(z_ref, w_ref, q_ref, idx_ref, perp_ref, uniq_ref,
              sww_ref, wm2_ref, wf_ref, counts_ref):
    i = pl.program_id(0)
    nb = pl.num_programs(0)

    @pl.when(i == 0)
    def _prep():
        w = w_ref[...]
        sww_ref[...] = jax.lax.dot_general(
            jnp.ones((1, _D), jnp.float32), w * w, (((1,), (1,)), ((), ())))
        wm2_ref[...] = w * (-2.0)
        wf_ref[...] = w
        counts_ref[...] = jnp.zeros_like(counts_ref)

    zb = z_ref[...].reshape(_R, _D)

    szz = jnp.sum(zb * zb, axis=1, keepdims=True)
    scores2 = jax.lax.dot_general(zb, wm2_ref[...],
                                  (((1,), (1,)), ((), ())))
    dist = (szz + sww_ref[...]) + scores2

    dmin = jnp.min(dist, axis=1, keepdims=True)
    lane = jax.lax.broadcasted_iota(jnp.int32, dist.shape, 1)
    idx = jnp.min(jnp.where(dist == dmin, lane, _NE),
                  axis=1, keepdims=True)

    onehot = (lane == idx).astype(jnp.float32)
    qb = jax.lax.dot_general(onehot, wf_ref[...], (((1,), (0,)), ((), ())),
                             preferred_element_type=jnp.float32)
    q_ref[...] = (zb + (qb - zb)).reshape(_RB, _N, _D)

    iota_row = jax.lax.broadcasted_iota(
        jnp.int32, (1, _NE), 1).astype(jnp.float32)
    base = jax.lax.rem(i, _IB // _RB) * _RB
    for j in range(_RB):
        oh_j = jax.lax.slice(onehot, (j * _N, 0), ((j + 1) * _N, _NE))
        row = jax.lax.dot_general(iota_row, oh_j, (((1,), (1,)), ((), ())),
                                  preferred_element_type=jnp.float32)
        idx_ref[pl.ds(base + j, 1), :] = row.astype(jnp.int32)

    counts_ref[...] += jax.lax.dot_general(
        jnp.ones((1, _R), jnp.float32), onehot, (((1,), (0,)), ((), ())),
        preferred_element_type=jnp.float32)

    @pl.when(i == nb - 1)
    def _fin():
        counts = counts_ref[...]
        avg = counts * (1.0 / _NTOK)
        ent = jnp.sum(avg * jnp.log(avg + 1e-10), axis=(0, 1), keepdims=True)
        perp_ref[...] = jnp.exp(-ent)
        uniq_ref[...] = jnp.sum((counts > 0.0).astype(jnp.int32),
                                axis=(0, 1), keepdims=True)


def kernel(z, W):
    original_dtype = z.dtype
    zf = z.astype(jnp.float32)
    q, idx, perp, uniq = pl.pallas_call(
        _vq_block,
        grid=(_NB,),
        in_specs=[
            pl.BlockSpec((_RB, _N, _D), lambda i: (i, 0, 0)),
            pl.BlockSpec((_NE, _D), lambda i: (0, 0)),
        ],
        out_specs=[
            pl.BlockSpec((_RB, _N, _D), lambda i: (i, 0, 0)),
            pl.BlockSpec((_IB, _N), lambda i: (i // (_IB // _RB), 0)),
            pl.BlockSpec((1, 1), lambda i: (0, 0)),
            pl.BlockSpec((1, 1), lambda i: (0, 0)),
        ],
        out_shape=[
            jax.ShapeDtypeStruct((_B, _N, _D), jnp.float32),
            jax.ShapeDtypeStruct((_B, _N), jnp.int32),
            jax.ShapeDtypeStruct((1, 1), jnp.float32),
            jax.ShapeDtypeStruct((1, 1), jnp.int32),
        ],
        scratch_shapes=[
            pltpu.VMEM((1, _NE), jnp.float32),
            pltpu.VMEM((_NE, _D), jnp.float32),
            pltpu.VMEM((_NE, _D), jnp.float32),
            pltpu.VMEM((1, _NE), jnp.float32),
        ],
        compiler_params=pltpu.CompilerParams(
            dimension_semantics=("arbitrary",)),
    )(zf, W)
    quantized = q.astype(original_dtype)
    vq_loss = jnp.zeros((), jnp.float32)
    return (quantized, idx, vq_loss, perp.reshape(()), uniq.reshape(()))


# speedup vs baseline: 1.6054x; 1.3435x over previous
import jax
import jax.numpy as jnp
from jax.experimental import pallas as pl
from jax.experimental.pallas import tpu as pltpu

_NE = 512
_D = 64
_B = 64
_N = 1024
_NTOK = _B * _N
_IB = 8


def _vq_row(zt_ref, wt_ref, qt_ref, idx_ref, perp_ref, uniq_ref,
            sww_ref, wtm2_ref, counts_ref):
    i = pl.program_id(0)
    nb = pl.num_programs(0)

    @pl.when(i == 0)
    def _prep():
        wt = wt_ref[...]
        wtm2_ref[...] = wt * (-2.0)
        sww_ref[...] = jax.lax.dot_general(
            wt * wt, jnp.ones((_D, 1), jnp.float32),
            (((0,), (0,)), ((), ())))
        counts_ref[...] = jnp.zeros_like(counts_ref)

    zt = zt_ref[...].reshape(_D, _N)

    s = zt * zt
    for k in (32, 16, 8, 4, 2, 1):
        s = jax.lax.slice(s, (0, 0), (k, _N)) + jax.lax.slice(
            s, (k, 0), (2 * k, _N))
    szz = s

    scores2 = jax.lax.dot_general(wtm2_ref[...], zt,
                                  (((0,), (0,)), ((), ())))
    dist = (szz + sww_ref[...]) + scores2

    dmin = jnp.min(dist, axis=0, keepdims=True)
    code = jax.lax.broadcasted_iota(jnp.int32, dist.shape, 0)
    idx = jnp.min(jnp.where(dist == dmin, code, _NE),
                  axis=0, keepdims=True)

    onehot = (code == idx).astype(jnp.float32)
    qt = jax.lax.dot_general(wt_ref[...], onehot, (((1,), (0,)), ((), ())),
                             preferred_element_type=jnp.float32)
    qt_ref[...] = (zt + (qt - zt)).reshape(1, _D, _N)
    idx_ref[pl.ds(jax.lax.rem(i, _IB), 1), :] = idx

    counts_ref[...] += jax.lax.dot_general(
        onehot, jnp.ones((_N, 1), jnp.float32), (((1,), (0,)), ((), ())),
        preferred_element_type=jnp.float32)

    @pl.when(i == nb - 1)
    def _fin():
        counts = counts_ref[...]
        avg = counts * (1.0 / _NTOK)
        ent = jnp.sum(avg * jnp.log(avg + 1e-10), axis=(0, 1), keepdims=True)
        perp_ref[...] = jnp.exp(-ent)
        uniq_ref[...] = jnp.sum((counts > 0.0).astype(jnp.int32),
                                axis=(0, 1), keepdims=True)


def kernel(z, W):
    original_dtype = z.dtype
    zt = jnp.swapaxes(z.astype(jnp.float32), 1, 2)
    wt = jnp.swapaxes(W, 0, 1)
    qt, idx, perp, uniq = pl.pallas_call(
        _vq_row,
        grid=(_B,),
        in_specs=[
            pl.BlockSpec((1, _D, _N), lambda i: (i, 0, 0)),
            pl.BlockSpec((_D, _NE), lambda i: (0, 0)),
        ],
        out_specs=[
            pl.BlockSpec((1, _D, _N), lambda i: (i, 0, 0)),
            pl.BlockSpec((_IB, _N), lambda i: (i // _IB, 0)),
            pl.BlockSpec((1, 1), lambda i: (0, 0)),
            pl.BlockSpec((1, 1), lambda i: (0, 0)),
        ],
        out_shape=[
            jax.ShapeDtypeStruct((_B, _D, _N), jnp.float32),
            jax.ShapeDtypeStruct((_B, _N), jnp.int32),
            jax.ShapeDtypeStruct((1, 1), jnp.float32),
            jax.ShapeDtypeStruct((1, 1), jnp.int32),
        ],
        scratch_shapes=[
            pltpu.VMEM((_NE, 1), jnp.float32),
            pltpu.VMEM((_D, _NE), jnp.float32),
            pltpu.VMEM((_NE, 1), jnp.float32),
        ],
        compiler_params=pltpu.CompilerParams(
            dimension_semantics=("arbitrary",)),
    )(zt, wt)
    quantized = jnp.swapaxes(qt, 1, 2).astype(original_dtype)
    vq_loss = jnp.zeros((), jnp.float32)
    return (quantized, idx, vq_loss, perp.reshape(()), uniq.reshape(()))
